# baseline (speedup 1.0000x reference)
import functools

import jax
import jax.numpy as jnp
from jax import lax
from jax.experimental import pallas as pl
from jax.experimental.pallas import tpu as pltpu
from jax.experimental.pallas import tpu_sc as plsc

_B = 16384
_L = 20
_NW = 32
_ROWS_W = _B // _NW
_GW = 64
_NGW = _ROWS_W // _GW


def _detile_body(x_ref, o_ref):
    o_ref[...] = x_ref[...].reshape(_L * _B // 128, 128)


_detile = pl.pallas_call(
    _detile_body,
    in_specs=[pl.BlockSpec((_L, _B), lambda: (0, 0))],
    out_specs=pl.BlockSpec((_L * _B // 128, 128), lambda: (0, 0)),
    out_shape=jax.ShapeDtypeStruct((_L * _B // 128, 128), jnp.int32),
)


def _sc_body(wid_lin, c1i, c2i, c3i, wemb, c1e, c2e, c3e,
             pack_out, cnt_out, c1t_out, c2t_out, c3t_out,
             idsv, cntv, rowsbuf, out_v, cidsv, cidx2, celems, sem):
    w = lax.axis_index("c") * 16 + lax.axis_index("s")
    base = w * _ROWS_W

    for l in range(_L):
        pltpu.sync_copy(wid_lin.at[pl.ds(l * 128 + w * 4, 4)], idsv.at[l])

    def kbody(j, carry):
        acc = jnp.zeros((16,), jnp.float32)
        for l in range(_L):
            acc = acc + jnp.where(
                idsv[l, j // 8, pl.ds((j % 8) * 16, 16)] != 0, 1.0, 0.0)
        cntv[pl.ds(j * 16, 16)] = jnp.maximum(acc, 1.0)
        return carry

    lax.fori_loop(0, _ROWS_W // 16, kbody, 0)
    pltpu.sync_copy(cntv, cnt_out.at[pl.ds(base, _ROWS_W)])

    for ids_h, emb_h, nrow, out_h in ((c1i, c1e, 1001, c1t_out),
                                      (c2i, c2e, 100001, c2t_out),
                                      (c3i, c3e, 1000001, c3t_out)):
        pltpu.sync_copy(ids_h.at[pl.ds(base, _ROWS_W)], cidsv)

        def ibody(k, carry):
            v = cidsv[pl.ds(k * 16, 16)]
            for c in range(16):
                cidx2[c, pl.ds(k * 16, 16)] = v + (c * nrow)
            return carry

        lax.fori_loop(0, _ROWS_W // 16, ibody, 0)

        def cgather(c, carry):
            cps = [pltpu.async_copy(
                       emb_h.at[cidx2.at[c, pl.ds(j * 128, 128)]],
                       celems.at[c, pl.ds(j * 128, 128)], sem)
                   for j in range(_ROWS_W // 128)]
            for cp in cps:
                cp.wait()
            return carry

        lax.fori_loop(0, 16, cgather, 0)
        pltpu.sync_copy(celems, out_h.at[:, pl.ds(base, _ROWS_W)])

    def gbody(g, carry):
        cps = [pltpu.async_copy(
                   wemb.at[idsv.at[l, g // 2, pl.ds((g % 2) * 64, 64)]],
                   rowsbuf.at[l], sem)
               for l in range(_L)]
        for cp in cps:
            cp.wait()

        def rbody(r, c2):
            for f in range(4):
                acc = rowsbuf[0, r, pl.ds(f * 16, 16)]
                for l in range(1, _L):
                    acc = acc + rowsbuf[l, r, pl.ds(f * 16, 16)]
                out_v[r, pl.ds(f * 16, 16)] = acc
            return c2

        lax.fori_loop(0, _GW, rbody, 0)
        pltpu.sync_copy(out_v,
                        pack_out.at[pl.ds(base + g * _GW, _GW), pl.ds(0, 64)])
        return carry

    lax.fori_loop(0, _NGW, gbody, 0)


_sc_gather = functools.partial(
    pl.kernel,
    out_type=[
        jax.ShapeDtypeStruct((_B, 128), jnp.float32),
        jax.ShapeDtypeStruct((_B,), jnp.float32),
        jax.ShapeDtypeStruct((16, _B), jnp.float32),
        jax.ShapeDtypeStruct((16, _B), jnp.float32),
        jax.ShapeDtypeStruct((16, _B), jnp.float32),
    ],
    mesh=plsc.VectorSubcoreMesh(core_axis_name="c", subcore_axis_name="s"),
    compiler_params=pltpu.CompilerParams(use_tc_tiling_on_sc=False),
    scratch_types=[
        pltpu.VMEM((_L, 4, 128), jnp.int32),
        pltpu.VMEM((_ROWS_W,), jnp.float32),
        pltpu.VMEM((_L, _GW, 64), jnp.float32),
        pltpu.VMEM((_GW, 64), jnp.float32),
        pltpu.VMEM((_ROWS_W,), jnp.int32),
        pltpu.VMEM((16, _ROWS_W), jnp.int32),
        pltpu.VMEM((16, _ROWS_W), jnp.float32),
        pltpu.SemaphoreType.DMA,
    ],
)(_sc_body)


def _ln(x, g, b, eps=1e-5):
    m = jnp.mean(x, axis=-1, keepdims=True)
    v = jnp.mean((x - m) ** 2, axis=-1, keepdims=True)
    return (x - m) / jnp.sqrt(v + eps) * g + b


def _gelu(x):
    return 0.5 * x * (1.0 + lax.erf(x * 0.7071067811865476))


_BLK = 1024


def _dgt(a, b):
    return lax.dot_general(a, b, (((0,), (0,)), ((), ())),
                           preferred_element_type=jnp.float32)


def _tc_body(pack_ref, cnt_ref, c1t_ref, c2t_ref, c3t_ref, numt_ref,
             tpw, tpb, tlg, tlb, npw, npb, nlg, nlb,
             f1w, f1b, flg, flb, f2w, f2b, out_ref):
    ccol = _dgt(cnt_ref[...], jnp.ones((1, 1), jnp.float32))
    mean = pack_ref[:, 0:64] / ccol
    t = _gelu(_ln(jnp.dot(mean, tpw[...],
                          preferred_element_type=jnp.float32) + tpb[...],
                  tlg[...], tlb[...]))
    n = _gelu(_ln(_dgt(numt_ref[...], npw[...]) + npb[...],
                  nlg[...], nlb[...]))
    f1 = f1w[...]
    hlin = (jnp.dot(t, f1[0:64, :], preferred_element_type=jnp.float32)
            + _dgt(c1t_ref[...], f1[64:80, :])
            + _dgt(c2t_ref[...], f1[80:96, :])
            + _dgt(c3t_ref[...], f1[96:112, :])
            + jnp.dot(n, f1[112:128, :], preferred_element_type=jnp.float32))
    h = _gelu(_ln(hlin + f1b[...], flg[...], flb[...]))
    out_ref[...] = jnp.dot(h, f2w[...],
                           preferred_element_type=jnp.float32) + f2b[...]


def _full(shape):
    return pl.BlockSpec(shape, lambda i: (0,) * len(shape))


_tc_encode = pl.pallas_call(
    _tc_body,
    grid=(_B // _BLK,),
    in_specs=[
        pl.BlockSpec((_BLK, 128), lambda i: (i, 0)),
        pl.BlockSpec((1, _BLK), lambda i: (0, i)),
        pl.BlockSpec((16, _BLK), lambda i: (0, i)),
        pl.BlockSpec((16, _BLK), lambda i: (0, i)),
        pl.BlockSpec((16, _BLK), lambda i: (0, i)),
        pl.BlockSpec((2, _BLK), lambda i: (0, i)),
        _full((64, 64)), _full((64,)), _full((64,)), _full((64,)),
        _full((2, 16)), _full((16,)), _full((16,)), _full((16,)),
        _full((128, 128)), _full((128,)), _full((128,)), _full((128,)),
        _full((128, 128)), _full((128,)),
    ],
    out_specs=pl.BlockSpec((_BLK, 128), lambda i: (i, 0)),
    out_shape=jax.ShapeDtypeStruct((_B, 128), jnp.float32),
)


def kernel(word_ids, cat1_ids, cat2_ids, cat3_ids, numerics,
           word_emb, cat1_emb, cat2_emb, cat3_emb,
           t_proj_w, t_proj_b, t_ln_g, t_ln_b,
           n_proj_w, n_proj_b, n_ln_g, n_ln_b,
           f1_w, f1_b, f_ln_g, f_ln_b, f2_w, f2_b):
    wid_lin = _detile(word_ids.T)
    pack, cnt, c1t, c2t, c3t = _sc_gather(
        wid_lin, cat1_ids, cat2_ids, cat3_ids,
        word_emb, cat1_emb.T.reshape(-1),
        cat2_emb.T.reshape(-1), cat3_emb.T.reshape(-1))
    return _tc_encode(pack, cnt.reshape(1, _B), c1t, c2t, c3t, numerics.T,
                      t_proj_w.T, t_proj_b, t_ln_g, t_ln_b,
                      n_proj_w.T, n_proj_b, n_ln_g, n_ln_b,
                      f1_w.T, f1_b, f_ln_g, f_ln_b,
                      f2_w.T, f2_b)


# speedup vs baseline: 1.9042x; 1.9042x over previous
import functools

import jax
import jax.numpy as jnp
from jax import lax
from jax.experimental import pallas as pl
from jax.experimental.pallas import tpu as pltpu
from jax.experimental.pallas import tpu_sc as plsc

_B = 16384
_L = 20
_NW = 32
_ROWS_W = _B // _NW
_GW = 64
_NGW = _ROWS_W // _GW


def _repack_body(x_ref, o_ref):
    o_ref[...] = x_ref[...].reshape(128, 128)


def _make_repack(nrow):
    blocks = (nrow + 1023) // 1024
    f = pl.pallas_call(
        _repack_body,
        grid=(blocks,),
        in_specs=[pl.BlockSpec((16, 1024), lambda i: (0, i))],
        out_specs=pl.BlockSpec((128, 128), lambda i: (i, 0)),
        out_shape=jax.ShapeDtypeStruct((blocks * 128, 128), jnp.float32),
    )
    return lambda emb: f(emb.T).reshape(-1)


_repack1 = _make_repack(1001)
_repack2 = _make_repack(100001)
_repack3 = _make_repack(1000001)


def _detile_body(x_ref, o_ref):
    o_ref[...] = x_ref[...].reshape(_L * _B // 128, 128)


_detile = pl.pallas_call(
    _detile_body,
    in_specs=[pl.BlockSpec((_L, _B), lambda: (0, 0))],
    out_specs=pl.BlockSpec((_L * _B // 128, 128), lambda: (0, 0)),
    out_shape=jax.ShapeDtypeStruct((_L * _B // 128, 128), jnp.int32),
)


def _sc_body(wid_lin, c1i, c2i, c3i, wemb, c1e, c2e, c3e,
             pack_out, cnt_out, c1t_out, c2t_out, c3t_out,
             idsv, cntv, rowsbuf, out_v, cidsv, cidx2, celems, sem):
    w = lax.axis_index("c") * 16 + lax.axis_index("s")
    base = w * _ROWS_W

    for l in range(_L):
        pltpu.sync_copy(wid_lin.at[pl.ds(l * 128 + w * 4, 4)], idsv.at[l])

    def kbody(j, carry):
        acc = jnp.zeros((16,), jnp.float32)
        for l in range(_L):
            acc = acc + jnp.where(
                idsv[l, j // 8, pl.ds((j % 8) * 16, 16)] != 0, 1.0, 0.0)
        cntv[pl.ds(j * 16, 16)] = jnp.maximum(acc, 1.0)
        return carry

    lax.fori_loop(0, _ROWS_W // 16, kbody, 0)
    pltpu.sync_copy(cntv, cnt_out.at[pl.ds(base, _ROWS_W)])

    for ids_h, emb_h, out_h in ((c1i, c1e, c1t_out),
                                (c2i, c2e, c2t_out),
                                (c3i, c3e, c3t_out)):
        pltpu.sync_copy(ids_h.at[pl.ds(base, _ROWS_W)], cidsv)

        def ibody(k, carry):
            v = cidsv[pl.ds(k * 16, 16)]
            b = ((v >> 10) << 14) + (v & 1023)
            for c in range(16):
                cidx2[c, pl.ds(k * 16, 16)] = b + (c * 1024)
            return carry

        lax.fori_loop(0, _ROWS_W // 16, ibody, 0)

        def cgather(c, carry):
            cps = [pltpu.async_copy(
                       emb_h.at[cidx2.at[c, pl.ds(j * 128, 128)]],
                       celems.at[c, pl.ds(j * 128, 128)], sem)
                   for j in range(_ROWS_W // 128)]
            for cp in cps:
                cp.wait()
            return carry

        lax.fori_loop(0, 16, cgather, 0)
        pltpu.sync_copy(celems, out_h.at[:, pl.ds(base, _ROWS_W)])

    def gbody(g, carry):
        cps = [pltpu.async_copy(
                   wemb.at[idsv.at[l, g // 2, pl.ds((g % 2) * 64, 64)]],
                   rowsbuf.at[l], sem)
               for l in range(_L)]
        for cp in cps:
            cp.wait()

        def rbody(r, c2):
            for f in range(4):
                acc = rowsbuf[0, r, pl.ds(f * 16, 16)]
                for l in range(1, _L):
                    acc = acc + rowsbuf[l, r, pl.ds(f * 16, 16)]
                out_v[r, pl.ds(f * 16, 16)] = acc
            return c2

        lax.fori_loop(0, _GW, rbody, 0)
        pltpu.sync_copy(out_v,
                        pack_out.at[pl.ds(base + g * _GW, _GW), pl.ds(0, 64)])
        return carry

    lax.fori_loop(0, _NGW, gbody, 0)


_sc_gather = functools.partial(
    pl.kernel,
    out_type=[
        jax.ShapeDtypeStruct((_B, 128), jnp.float32),
        jax.ShapeDtypeStruct((_B,), jnp.float32),
        jax.ShapeDtypeStruct((16, _B), jnp.float32),
        jax.ShapeDtypeStruct((16, _B), jnp.float32),
        jax.ShapeDtypeStruct((16, _B), jnp.float32),
    ],
    mesh=plsc.VectorSubcoreMesh(core_axis_name="c", subcore_axis_name="s"),
    compiler_params=pltpu.CompilerParams(use_tc_tiling_on_sc=False),
    scratch_types=[
        pltpu.VMEM((_L, 4, 128), jnp.int32),
        pltpu.VMEM((_ROWS_W,), jnp.float32),
        pltpu.VMEM((_L, _GW, 64), jnp.float32),
        pltpu.VMEM((_GW, 64), jnp.float32),
        pltpu.VMEM((_ROWS_W,), jnp.int32),
        pltpu.VMEM((16, _ROWS_W), jnp.int32),
        pltpu.VMEM((16, _ROWS_W), jnp.float32),
        pltpu.SemaphoreType.DMA,
    ],
)(_sc_body)


def _ln(x, g, b, eps=1e-5):
    m = jnp.mean(x, axis=-1, keepdims=True)
    v = jnp.mean((x - m) ** 2, axis=-1, keepdims=True)
    return (x - m) / jnp.sqrt(v + eps) * g + b


def _gelu(x):
    return 0.5 * x * (1.0 + lax.erf(x * 0.7071067811865476))


_BLK = 1024


def _dgt(a, b):
    return lax.dot_general(a, b, (((0,), (0,)), ((), ())),
                           preferred_element_type=jnp.float32)


def _tc_body(pack_ref, cnt_ref, c1t_ref, c2t_ref, c3t_ref, numt_ref,
             tpw, tpb, tlg, tlb, npw, npb, nlg, nlb,
             f1w, f1b, flg, flb, f2w, f2b, out_ref):
    ccol = _dgt(cnt_ref[...], jnp.ones((1, 1), jnp.float32))
    mean = pack_ref[:, 0:64] / ccol
    t = _gelu(_ln(jnp.dot(mean, tpw[...],
                          preferred_element_type=jnp.float32) + tpb[...],
                  tlg[...], tlb[...]))
    n = _gelu(_ln(_dgt(numt_ref[...], npw[...]) + npb[...],
                  nlg[...], nlb[...]))
    f1 = f1w[...]
    hlin = (jnp.dot(t, f1[0:64, :], preferred_element_type=jnp.float32)
            + _dgt(c1t_ref[...], f1[64:80, :])
            + _dgt(c2t_ref[...], f1[80:96, :])
            + _dgt(c3t_ref[...], f1[96:112, :])
            + jnp.dot(n, f1[112:128, :], preferred_element_type=jnp.float32))
    h = _gelu(_ln(hlin + f1b[...], flg[...], flb[...]))
    out_ref[...] = jnp.dot(h, f2w[...],
                           preferred_element_type=jnp.float32) + f2b[...]


def _full(shape):
    return pl.BlockSpec(shape, lambda i: (0,) * len(shape))


_tc_encode = pl.pallas_call(
    _tc_body,
    grid=(_B // _BLK,),
    in_specs=[
        pl.BlockSpec((_BLK, 128), lambda i: (i, 0)),
        pl.BlockSpec((1, _BLK), lambda i: (0, i)),
        pl.BlockSpec((16, _BLK), lambda i: (0, i)),
        pl.BlockSpec((16, _BLK), lambda i: (0, i)),
        pl.BlockSpec((16, _BLK), lambda i: (0, i)),
        pl.BlockSpec((2, _BLK), lambda i: (0, i)),
        _full((64, 64)), _full((64,)), _full((64,)), _full((64,)),
        _full((2, 16)), _full((16,)), _full((16,)), _full((16,)),
        _full((128, 128)), _full((128,)), _full((128,)), _full((128,)),
        _full((128, 128)), _full((128,)),
    ],
    out_specs=pl.BlockSpec((_BLK, 128), lambda i: (i, 0)),
    out_shape=jax.ShapeDtypeStruct((_B, 128), jnp.float32),
)


def kernel(word_ids, cat1_ids, cat2_ids, cat3_ids, numerics,
           word_emb, cat1_emb, cat2_emb, cat3_emb,
           t_proj_w, t_proj_b, t_ln_g, t_ln_b,
           n_proj_w, n_proj_b, n_ln_g, n_ln_b,
           f1_w, f1_b, f_ln_g, f_ln_b, f2_w, f2_b):
    wid_lin = _detile(word_ids.T)
    pack, cnt, c1t, c2t, c3t = _sc_gather(
        wid_lin, cat1_ids, cat2_ids, cat3_ids,
        word_emb, _repack1(cat1_emb), _repack2(cat2_emb),
        _repack3(cat3_emb))
    return _tc_encode(pack, cnt.reshape(1, _B), c1t, c2t, c3t, numerics.T,
                      t_proj_w.T, t_proj_b, t_ln_g, t_ln_b,
                      n_proj_w.T, n_proj_b, n_ln_g, n_ln_b,
                      f1_w.T, f1_b, f_ln_g, f_ln_b,
                      f2_w.T, f2_b)


# speedup vs baseline: 4.8084x; 2.5252x over previous
import functools

import jax
import jax.numpy as jnp
from jax import lax
from jax.experimental import pallas as pl
from jax.experimental.pallas import tpu as pltpu
from jax.experimental.pallas import tpu_sc as plsc

_B = 16384
_L = 20
_NW = 32
_ROWS_W = _B // _NW
_GW = 64
_NGW = _ROWS_W // _GW


_RW = 16384


def _repack_body(x_ref, o_ref):
    o_ref[...] = x_ref[...].reshape(16 * _RW // 128, 128)


def _make_repack(nrow):
    blocks = (nrow + _RW - 1) // _RW
    f = pl.pallas_call(
        _repack_body,
        grid=(blocks,),
        in_specs=[pl.BlockSpec((16, _RW), lambda i: (0, i))],
        out_specs=pl.BlockSpec((16 * _RW // 128, 128), lambda i: (i, 0)),
        out_shape=jax.ShapeDtypeStruct((blocks * 16 * _RW // 128, 128),
                                       jnp.float32),
    )
    return lambda emb: f(emb.T).reshape(-1)


_repack1 = _make_repack(1001)
_repack2 = _make_repack(100001)
_repack3 = _make_repack(1000001)


def _detile_body(x_ref, o_ref):
    o_ref[...] = x_ref[...].reshape(_L * _B // 128, 128)


_detile = pl.pallas_call(
    _detile_body,
    in_specs=[pl.BlockSpec((_L, _B), lambda: (0, 0))],
    out_specs=pl.BlockSpec((_L * _B // 128, 128), lambda: (0, 0)),
    out_shape=jax.ShapeDtypeStruct((_L * _B // 128, 128), jnp.int32),
)


def _sc_body(wid_lin, c1i, c2i, c3i, wemb, c1e, c2e, c3e,
             pack_out, cnt_out, c1t_out, c2t_out, c3t_out,
             idsv, cntv, rowsbuf, out_v, cidsv, cidx2, celems, sem):
    w = lax.axis_index("c") * 16 + lax.axis_index("s")
    base = w * _ROWS_W

    for l in range(_L):
        pltpu.sync_copy(wid_lin.at[pl.ds(l * 128 + w * 4, 4)], idsv.at[l])

    def kbody(j, carry):
        acc = jnp.zeros((16,), jnp.float32)
        for l in range(_L):
            acc = acc + jnp.where(
                idsv[l, j // 8, pl.ds((j % 8) * 16, 16)] != 0, 1.0, 0.0)
        cntv[pl.ds(j * 16, 16)] = jnp.maximum(acc, 1.0)
        return carry

    lax.fori_loop(0, _ROWS_W // 16, kbody, 0)
    pltpu.sync_copy(cntv, cnt_out.at[pl.ds(base, _ROWS_W)])

    for ids_h, emb_h, out_h in ((c1i, c1e, c1t_out),
                                (c2i, c2e, c2t_out),
                                (c3i, c3e, c3t_out)):
        pltpu.sync_copy(ids_h.at[pl.ds(base, _ROWS_W)], cidsv)

        def ibody(k, carry):
            v = cidsv[pl.ds(k * 16, 16)]
            b = ((v >> 14) << 18) + (v & (_RW - 1))
            for c in range(16):
                cidx2[c, pl.ds(k * 16, 16)] = b + (c * _RW)
            return carry

        lax.fori_loop(0, _ROWS_W // 16, ibody, 0)

        def cgather(c, carry):
            cps = [pltpu.async_copy(
                       emb_h.at[cidx2.at[c, pl.ds(j * 128, 128)]],
                       celems.at[c, pl.ds(j * 128, 128)], sem)
                   for j in range(_ROWS_W // 128)]
            for cp in cps:
                cp.wait()
            return carry

        lax.fori_loop(0, 16, cgather, 0)
        pltpu.sync_copy(celems, out_h.at[:, pl.ds(base, _ROWS_W)])

    def gbody(g, carry):
        cps = [pltpu.async_copy(
                   wemb.at[idsv.at[l, g // 2, pl.ds((g % 2) * 64, 64)]],
                   rowsbuf.at[l], sem)
               for l in range(_L)]
        for cp in cps:
            cp.wait()

        def rbody(r, c2):
            for f in range(4):
                acc = rowsbuf[0, r, pl.ds(f * 16, 16)]
                for l in range(1, _L):
                    acc = acc + rowsbuf[l, r, pl.ds(f * 16, 16)]
                out_v[r, pl.ds(f * 16, 16)] = acc
            return c2

        lax.fori_loop(0, _GW, rbody, 0)
        pltpu.sync_copy(out_v,
                        pack_out.at[pl.ds(base + g * _GW, _GW), pl.ds(0, 64)])
        return carry

    lax.fori_loop(0, _NGW, gbody, 0)


_sc_gather = functools.partial(
    pl.kernel,
    out_type=[
        jax.ShapeDtypeStruct((_B, 128), jnp.float32),
        jax.ShapeDtypeStruct((_B,), jnp.float32),
        jax.ShapeDtypeStruct((16, _B), jnp.float32),
        jax.ShapeDtypeStruct((16, _B), jnp.float32),
        jax.ShapeDtypeStruct((16, _B), jnp.float32),
    ],
    mesh=plsc.VectorSubcoreMesh(core_axis_name="c", subcore_axis_name="s"),
    compiler_params=pltpu.CompilerParams(use_tc_tiling_on_sc=False),
    scratch_types=[
        pltpu.VMEM((_L, 4, 128), jnp.int32),
        pltpu.VMEM((_ROWS_W,), jnp.float32),
        pltpu.VMEM((_L, _GW, 64), jnp.float32),
        pltpu.VMEM((_GW, 64), jnp.float32),
        pltpu.VMEM((_ROWS_W,), jnp.int32),
        pltpu.VMEM((16, _ROWS_W), jnp.int32),
        pltpu.VMEM((16, _ROWS_W), jnp.float32),
        pltpu.SemaphoreType.DMA,
    ],
)(_sc_body)


def _ln(x, g, b, eps=1e-5):
    m = jnp.mean(x, axis=-1, keepdims=True)
    v = jnp.mean((x - m) ** 2, axis=-1, keepdims=True)
    return (x - m) / jnp.sqrt(v + eps) * g + b


def _gelu(x):
    return 0.5 * x * (1.0 + lax.erf(x * 0.7071067811865476))


_BLK = 1024


def _dgt(a, b):
    return lax.dot_general(a, b, (((0,), (0,)), ((), ())),
                           preferred_element_type=jnp.float32)


def _tc_body(pack_ref, cnt_ref, c1t_ref, c2t_ref, c3t_ref, numt_ref,
             tpw, tpb, tlg, tlb, npw, npb, nlg, nlb,
             f1w, f1b, flg, flb, f2w, f2b, out_ref):
    ccol = _dgt(cnt_ref[...], jnp.ones((1, 1), jnp.float32))
    mean = pack_ref[:, 0:64] / ccol
    t = _gelu(_ln(jnp.dot(mean, tpw[...],
                          preferred_element_type=jnp.float32) + tpb[...],
                  tlg[...], tlb[...]))
    n = _gelu(_ln(_dgt(numt_ref[...], npw[...]) + npb[...],
                  nlg[...], nlb[...]))
    f1 = f1w[...]
    hlin = (jnp.dot(t, f1[0:64, :], preferred_element_type=jnp.float32)
            + _dgt(c1t_ref[...], f1[64:80, :])
            + _dgt(c2t_ref[...], f1[80:96, :])
            + _dgt(c3t_ref[...], f1[96:112, :])
            + jnp.dot(n, f1[112:128, :], preferred_element_type=jnp.float32))
    h = _gelu(_ln(hlin + f1b[...], flg[...], flb[...]))
    out_ref[...] = jnp.dot(h, f2w[...],
                           preferred_element_type=jnp.float32) + f2b[...]


def _full(shape):
    return pl.BlockSpec(shape, lambda i: (0,) * len(shape))


_tc_encode = pl.pallas_call(
    _tc_body,
    grid=(_B // _BLK,),
    in_specs=[
        pl.BlockSpec((_BLK, 128), lambda i: (i, 0)),
        pl.BlockSpec((1, _BLK), lambda i: (0, i)),
        pl.BlockSpec((16, _BLK), lambda i: (0, i)),
        pl.BlockSpec((16, _BLK), lambda i: (0, i)),
        pl.BlockSpec((16, _BLK), lambda i: (0, i)),
        pl.BlockSpec((2, _BLK), lambda i: (0, i)),
        _full((64, 64)), _full((64,)), _full((64,)), _full((64,)),
        _full((2, 16)), _full((16,)), _full((16,)), _full((16,)),
        _full((128, 128)), _full((128,)), _full((128,)), _full((128,)),
        _full((128, 128)), _full((128,)),
    ],
    out_specs=pl.BlockSpec((_BLK, 128), lambda i: (i, 0)),
    out_shape=jax.ShapeDtypeStruct((_B, 128), jnp.float32),
)


def kernel(word_ids, cat1_ids, cat2_ids, cat3_ids, numerics,
           word_emb, cat1_emb, cat2_emb, cat3_emb,
           t_proj_w, t_proj_b, t_ln_g, t_ln_b,
           n_proj_w, n_proj_b, n_ln_g, n_ln_b,
           f1_w, f1_b, f_ln_g, f_ln_b, f2_w, f2_b):
    wid_lin = _detile(word_ids.T)
    pack, cnt, c1t, c2t, c3t = _sc_gather(
        wid_lin, cat1_ids, cat2_ids, cat3_ids,
        word_emb, _repack1(cat1_emb), _repack2(cat2_emb),
        _repack3(cat3_emb))
    return _tc_encode(pack, cnt.reshape(1, _B), c1t, c2t, c3t, numerics.T,
                      t_proj_w.T, t_proj_b, t_ln_g, t_ln_b,
                      n_proj_w.T, n_proj_b, n_ln_g, n_ln_b,
                      f1_w.T, f1_b, f_ln_g, f_ln_b,
                      f2_w.T, f2_b)
